# baseline (speedup 1.0000x reference)
import functools

import jax
import jax.numpy as jnp
from jax.experimental import pallas as pl

N = 2048
BM = 256
_PREC = jax.lax.Precision.HIGHEST


def _rows(i):
    return (i, 0)


def _const(i):
    return (0, 0)


def _k1_body(x_ref, we1_ref, be1_ref, wg1_ref, a1s_ref, a1d_ref,
             enc_ref, wh_ref, fs_ref, fd_ref):
    xb = x_ref[...]
    enc = jnp.dot(xb, we1_ref[...], precision=_PREC,
                  preferred_element_type=jnp.float32) + be1_ref[...]
    enc_ref[...] = jnp.maximum(enc, 0.0)
    wh = jnp.dot(xb, wg1_ref[...], precision=_PREC,
                 preferred_element_type=jnp.float32)
    wh_ref[...] = wh
    fs_ref[...] = jnp.sum(wh * a1s_ref[...], axis=1, keepdims=True)
    fd_ref[...] = jnp.sum(wh * a1d_ref[...], axis=1, keepdims=True)


def _k1(x_pad, We1p, be1p, Wg1p, a1s_row, a1d_row):
    dp = x_pad.shape[1]
    e1 = We1p.shape[1]
    return pl.pallas_call(
        _k1_body,
        grid=(N // BM,),
        in_specs=[
            pl.BlockSpec((BM, dp), _rows),
            pl.BlockSpec((dp, e1), _const),
            pl.BlockSpec((1, e1), _const),
            pl.BlockSpec((dp, e1), _const),
            pl.BlockSpec((1, e1), _const),
            pl.BlockSpec((1, e1), _const),
        ],
        out_specs=[
            pl.BlockSpec((BM, e1), _rows),
            pl.BlockSpec((BM, e1), _rows),
            pl.BlockSpec((BM, 1), _rows),
            pl.BlockSpec((BM, 1), _rows),
        ],
        out_shape=[
            jax.ShapeDtypeStruct((N, e1), jnp.float32),
            jax.ShapeDtypeStruct((N, e1), jnp.float32),
            jax.ShapeDtypeStruct((N, 1), jnp.float32),
            jax.ShapeDtypeStruct((N, 1), jnp.float32),
        ],
    )(x_pad, We1p, be1p, Wg1p, a1s_row, a1d_row)


def _k2_body(enc_ref, we2_ref, be2_ref, wz_ref, bz_ref, wd1_ref, bd1_ref,
             wd2_ref, bd2_ref, wxb_ref, bxb_ref, h2_ref, z_ref, xb_ref):
    h1 = enc_ref[...]
    h2 = jnp.maximum(jnp.dot(h1, we2_ref[...], precision=_PREC,
                             preferred_element_type=jnp.float32)
                     + be2_ref[...], 0.0)
    h2_ref[...] = h2
    z = jnp.dot(h2, wz_ref[...], precision=_PREC,
                preferred_element_type=jnp.float32) + bz_ref[...]
    z_ref[...] = z
    d1 = jnp.maximum(jnp.dot(z, wd1_ref[...], precision=_PREC,
                             preferred_element_type=jnp.float32)
                     + bd1_ref[...], 0.0)
    d2 = jnp.maximum(jnp.dot(d1, wd2_ref[...], precision=_PREC,
                             preferred_element_type=jnp.float32)
                     + bd2_ref[...], 0.0)
    xb_ref[...] = jnp.dot(d2, wxb_ref[...], precision=_PREC,
                          preferred_element_type=jnp.float32) + bxb_ref[...]


def _k2(enc_h1, We2p, be2, Wz, bz, Wd1, bd1, Wd2p, bd2p, Wxbp, bxbp):
    e1 = enc_h1.shape[1]
    e2 = We2p.shape[1]
    nz = Wz.shape[1]
    dp = Wxbp.shape[1]
    return pl.pallas_call(
        _k2_body,
        grid=(N // BM,),
        in_specs=[
            pl.BlockSpec((BM, e1), _rows),
            pl.BlockSpec((e1, e2), _const),
            pl.BlockSpec((1, e2), _const),
            pl.BlockSpec((e2, nz), _const),
            pl.BlockSpec((1, nz), _const),
            pl.BlockSpec((nz, e2), _const),
            pl.BlockSpec((1, e2), _const),
            pl.BlockSpec((e2, e1), _const),
            pl.BlockSpec((1, e1), _const),
            pl.BlockSpec((e1, dp), _const),
            pl.BlockSpec((1, dp), _const),
        ],
        out_specs=[
            pl.BlockSpec((BM, e2), _rows),
            pl.BlockSpec((BM, nz), _rows),
            pl.BlockSpec((BM, dp), _rows),
        ],
        out_shape=[
            jax.ShapeDtypeStruct((N, e2), jnp.float32),
            jax.ShapeDtypeStruct((N, nz), jnp.float32),
            jax.ShapeDtypeStruct((N, dp), jnp.float32),
        ],
    )(enc_h1, We2p, be2, Wz, bz, Wd1, bd1, Wd2p, bd2p, Wxbp, bxbp)


def _gat_body(adj_ref, fs_ref, fd_ref, wh_ref, o_ref, *, elu):
    e = fs_ref[...] + fd_ref[...]
    e = jnp.where(e >= 0, e, 0.2 * e)
    masked = jnp.where(adj_ref[...] > 0, e, jnp.float32(-9e15))
    m = jnp.max(masked, axis=1, keepdims=True)
    p = jnp.exp(masked - m)
    s = jnp.sum(p, axis=1, keepdims=True)
    att = p / s
    h = jnp.dot(att, wh_ref[...], precision=_PREC,
                preferred_element_type=jnp.float32)
    if elu:
        h = jnp.where(h > 0, h, jnp.exp(h) - 1.0)
    o_ref[...] = h


def _gat(adj, fs, fd_row, Wh, elu):
    d = Wh.shape[1]
    return pl.pallas_call(
        functools.partial(_gat_body, elu=elu),
        grid=(N // BM,),
        in_specs=[
            pl.BlockSpec((BM, N), _rows),
            pl.BlockSpec((BM, 1), _rows),
            pl.BlockSpec((1, N), _const),
            pl.BlockSpec((N, d), _const),
        ],
        out_specs=pl.BlockSpec((BM, d), _rows),
        out_shape=jax.ShapeDtypeStruct((N, d), jnp.float32),
    )(adj, fs, fd_row, Wh)


def _mix_body(h_ref, t_ref, wg_ref, as_ref, ad_ref, wh_ref, fs_ref, fd_ref):
    mix = 0.5 * h_ref[...] + 0.5 * t_ref[...]
    wh = jnp.dot(mix, wg_ref[...], precision=_PREC,
                 preferred_element_type=jnp.float32)
    wh_ref[...] = wh
    fs_ref[...] = jnp.sum(wh * as_ref[...], axis=1, keepdims=True)
    fd_ref[...] = jnp.sum(wh * ad_ref[...], axis=1, keepdims=True)


def _mix_project(h, tra, Wg, as_row, ad_row):
    e = h.shape[1]
    f = Wg.shape[1]
    return pl.pallas_call(
        _mix_body,
        grid=(N // BM,),
        in_specs=[
            pl.BlockSpec((BM, e), _rows),
            pl.BlockSpec((BM, e), _rows),
            pl.BlockSpec((e, f), _const),
            pl.BlockSpec((1, f), _const),
            pl.BlockSpec((1, f), _const),
        ],
        out_specs=[
            pl.BlockSpec((BM, f), _rows),
            pl.BlockSpec((BM, 1), _rows),
            pl.BlockSpec((BM, 1), _rows),
        ],
        out_shape=[
            jax.ShapeDtypeStruct((N, f), jnp.float32),
            jax.ShapeDtypeStruct((N, 1), jnp.float32),
            jax.ShapeDtypeStruct((N, 1), jnp.float32),
        ],
    )(h, tra, Wg, as_row, ad_row)


def _fuse_body(z_ref, ha_ref, hk_ref, hd_ref, wp1_ref, bp1_ref, wp2_ref,
               emb_ref, ba_ref, bk_ref, bd_ref):
    wp1 = wp1_ref[...]
    bp1 = bp1_ref[...]
    wp2 = wp2_ref[...]

    def score(u):
        t = jnp.tanh(jnp.dot(u, wp1, precision=_PREC,
                             preferred_element_type=jnp.float32) + bp1)
        return jnp.sum(t * wp2, axis=1, keepdims=True)

    zb = z_ref[...]
    wz = score(zb)
    embs = []
    for h_ref, b_ref in ((ha_ref, ba_ref), (hk_ref, bk_ref), (hd_ref, bd_ref)):
        hb = h_ref[...]
        wh = score(hb)
        m = jnp.maximum(wh, wz)
        p1 = jnp.exp(wh - m)
        p2 = jnp.exp(wz - m)
        s = p1 + p2
        b1 = p1 / s
        b2 = p2 / s
        b_ref[...] = jnp.concatenate([b1, b2], axis=1)
        embs.append(b1 * hb + b2 * zb)

    w1, w2, w3 = score(embs[0]), score(embs[1]), score(embs[2])
    m = jnp.maximum(jnp.maximum(w1, w2), w3)
    p1 = jnp.exp(w1 - m)
    p2 = jnp.exp(w2 - m)
    p3 = jnp.exp(w3 - m)
    s = p1 + p2 + p3
    emb_ref[...] = (p1 / s) * embs[0] + (p2 / s) * embs[1] + (p3 / s) * embs[2]


def _fuse(z, h3a, h3k, h3d, Wp1, bp1_row, wp2_row):
    nz = z.shape[1]
    return pl.pallas_call(
        _fuse_body,
        grid=(N // BM,),
        in_specs=[
            pl.BlockSpec((BM, nz), _rows),
            pl.BlockSpec((BM, nz), _rows),
            pl.BlockSpec((BM, nz), _rows),
            pl.BlockSpec((BM, nz), _rows),
            pl.BlockSpec((nz, nz), _const),
            pl.BlockSpec((1, nz), _const),
            pl.BlockSpec((1, nz), _const),
        ],
        out_specs=[
            pl.BlockSpec((BM, nz), _rows),
            pl.BlockSpec((BM, 2), _rows),
            pl.BlockSpec((BM, 2), _rows),
            pl.BlockSpec((BM, 2), _rows),
        ],
        out_shape=[
            jax.ShapeDtypeStruct((N, nz), jnp.float32),
            jax.ShapeDtypeStruct((N, 2), jnp.float32),
            jax.ShapeDtypeStruct((N, 2), jnp.float32),
            jax.ShapeDtypeStruct((N, 2), jnp.float32),
        ],
    )(z, h3a, h3k, h3d, Wp1, bp1_row, wp2_row)


def _pad2(a, r, c):
    return jnp.pad(a, ((0, r - a.shape[0]), (0, c - a.shape[1])))


def _pad_row(v, c):
    return jnp.pad(v.reshape(1, -1), ((0, 0), (0, c - v.shape[0])))


def kernel(x, adj, adj_knn, adj_diff, We1, be1, We2, be2, Wz, bz, Wd1, bd1,
           Wd2, bd2, Wxb, bxb, Wg1, ag1s, ag1d, Wg2, ag2s, ag2d, Wg3, ag3s,
           ag3d, Wp1, bp1, Wp2):
    d_in = x.shape[1]
    dp = 1792
    e1p = 2048
    e2 = We2.shape[1]
    nz = Wz.shape[1]

    x_pad = _pad2(x, N, dp)
    We1p = _pad2(We1, dp, e1p)
    be1p = _pad_row(be1, e1p)
    Wg1p = _pad2(Wg1, dp, e1p)
    a1s_row = _pad_row(ag1s, e1p)
    a1d_row = _pad_row(ag1d, e1p)
    We2p = _pad2(We2, e1p, e2)
    Wd2p = _pad2(Wd2, e2, e1p)
    bd2p = _pad_row(bd2, e1p)
    Wxbp = _pad2(Wxb, e1p, dp)
    bxbp = _pad_row(bxb, dp)
    Wg2p = _pad2(Wg2, e1p, e2)

    enc_h1, Wh1, f1s, f1d = _k1(x_pad, We1p, be1p, Wg1p, a1s_row, a1d_row)
    f1d_row = f1d.reshape(1, N)

    enc_h2, z, xbar_pad = _k2(enc_h1, We2p, be2.reshape(1, -1), Wz,
                              bz.reshape(1, -1), Wd1, bd1.reshape(1, -1),
                              Wd2p, bd2p, Wxbp, bxbp)
    x_bar = xbar_pad[:, :d_in]

    a2s_row = ag2s.reshape(1, -1)
    a2d_row = ag2d.reshape(1, -1)
    a3s_row = ag3s.reshape(1, -1)
    a3d_row = ag3d.reshape(1, -1)

    h3 = {}
    for name, adj_12, adj_3 in (("adj", adj, adj),
                                ("knn", adj_knn, adj),
                                ("diff", adj_diff, adj_diff)):
        h1_v = _gat(adj_12, f1s, f1d_row, Wh1, elu=True)
        Wh2, f2s, f2d = _mix_project(h1_v, enc_h1, Wg2p, a2s_row, a2d_row)
        h2_v = _gat(adj_12, f2s, f2d.reshape(1, N), Wh2, elu=True)
        Wh3, f3s, f3d = _mix_project(h2_v, enc_h2, Wg3, a3s_row, a3d_row)
        h3[name] = _gat(adj_3, f3s, f3d.reshape(1, N), Wh3, elu=False)

    emb_last, b_adj, b_knn, b_diff = _fuse(
        z, h3["adj"], h3["knn"], h3["diff"], Wp1, bp1.reshape(1, -1),
        Wp2.reshape(1, -1))

    return (emb_last,
            b_adj.reshape(N, 2, 1),
            b_knn.reshape(N, 2, 1),
            b_diff.reshape(N, 2, 1),
            x_bar)


# speedup vs baseline: 1.0727x; 1.0727x over previous
import functools

import jax
import jax.numpy as jnp
from jax.experimental import pallas as pl

N = 2048
BM = 256
_PREC = jax.lax.Precision.HIGHEST


def _rows(i):
    return (i, 0)


def _const(i):
    return (0, 0)


def _k1_body(x_ref, we1_ref, be1_ref, wg1_ref, a1s_ref, a1d_ref,
             enc_ref, wh_ref, fs_ref, fd_ref):
    xb = x_ref[...]
    enc = jnp.dot(xb, we1_ref[...], precision=_PREC,
                  preferred_element_type=jnp.float32) + be1_ref[...]
    enc_ref[...] = jnp.maximum(enc, 0.0)
    wh = jnp.dot(xb, wg1_ref[...], precision=_PREC,
                 preferred_element_type=jnp.float32)
    wh_ref[...] = wh
    fs_ref[...] = jnp.sum(wh * a1s_ref[...], axis=1, keepdims=True)
    fd_ref[...] = jnp.sum(wh * a1d_ref[...], axis=1, keepdims=True)


def _k1(x_pad, We1p, be1p, Wg1p, a1s_row, a1d_row):
    dp = x_pad.shape[1]
    e1 = We1p.shape[1]
    return pl.pallas_call(
        _k1_body,
        grid=(N // BM,),
        in_specs=[
            pl.BlockSpec((BM, dp), _rows),
            pl.BlockSpec((dp, e1), _const),
            pl.BlockSpec((1, e1), _const),
            pl.BlockSpec((dp, e1), _const),
            pl.BlockSpec((1, e1), _const),
            pl.BlockSpec((1, e1), _const),
        ],
        out_specs=[
            pl.BlockSpec((BM, e1), _rows),
            pl.BlockSpec((BM, e1), _rows),
            pl.BlockSpec((BM, 1), _rows),
            pl.BlockSpec((BM, 1), _rows),
        ],
        out_shape=[
            jax.ShapeDtypeStruct((N, e1), jnp.float32),
            jax.ShapeDtypeStruct((N, e1), jnp.float32),
            jax.ShapeDtypeStruct((N, 1), jnp.float32),
            jax.ShapeDtypeStruct((N, 1), jnp.float32),
        ],
    )(x_pad, We1p, be1p, Wg1p, a1s_row, a1d_row)


def _k2_body(enc_ref, we2_ref, be2_ref, wz_ref, bz_ref, wd1_ref, bd1_ref,
             wd2_ref, bd2_ref, wxb_ref, bxb_ref, h2_ref, z_ref, xb_ref):
    h1 = enc_ref[...]
    h2 = jnp.maximum(jnp.dot(h1, we2_ref[...], precision=_PREC,
                             preferred_element_type=jnp.float32)
                     + be2_ref[...], 0.0)
    h2_ref[...] = h2
    z = jnp.dot(h2, wz_ref[...], precision=_PREC,
                preferred_element_type=jnp.float32) + bz_ref[...]
    z_ref[...] = z
    d1 = jnp.maximum(jnp.dot(z, wd1_ref[...], precision=_PREC,
                             preferred_element_type=jnp.float32)
                     + bd1_ref[...], 0.0)
    d2 = jnp.maximum(jnp.dot(d1, wd2_ref[...], precision=_PREC,
                             preferred_element_type=jnp.float32)
                     + bd2_ref[...], 0.0)
    xb_ref[...] = jnp.dot(d2, wxb_ref[...], precision=_PREC,
                          preferred_element_type=jnp.float32) + bxb_ref[...]


def _k2(enc_h1, We2p, be2, Wz, bz, Wd1, bd1, Wd2p, bd2p, Wxbp, bxbp):
    e1 = enc_h1.shape[1]
    e2 = We2p.shape[1]
    nz = Wz.shape[1]
    dp = Wxbp.shape[1]
    return pl.pallas_call(
        _k2_body,
        grid=(N // BM,),
        in_specs=[
            pl.BlockSpec((BM, e1), _rows),
            pl.BlockSpec((e1, e2), _const),
            pl.BlockSpec((1, e2), _const),
            pl.BlockSpec((e2, nz), _const),
            pl.BlockSpec((1, nz), _const),
            pl.BlockSpec((nz, e2), _const),
            pl.BlockSpec((1, e2), _const),
            pl.BlockSpec((e2, e1), _const),
            pl.BlockSpec((1, e1), _const),
            pl.BlockSpec((e1, dp), _const),
            pl.BlockSpec((1, dp), _const),
        ],
        out_specs=[
            pl.BlockSpec((BM, e2), _rows),
            pl.BlockSpec((BM, nz), _rows),
            pl.BlockSpec((BM, dp), _rows),
        ],
        out_shape=[
            jax.ShapeDtypeStruct((N, e2), jnp.float32),
            jax.ShapeDtypeStruct((N, nz), jnp.float32),
            jax.ShapeDtypeStruct((N, dp), jnp.float32),
        ],
    )(enc_h1, We2p, be2, Wz, bz, Wd1, bd1, Wd2p, bd2p, Wxbp, bxbp)


def _gat_body(adj_ref, fs_ref, fd_ref, wh_ref, o_ref, *, elu):
    e = fs_ref[...] + fd_ref[...]
    e = jnp.where(e >= 0, e, 0.2 * e)
    masked = jnp.where(adj_ref[...] > 0, e, jnp.float32(-9e15))
    m = jnp.max(masked, axis=1, keepdims=True)
    p = jnp.exp(masked - m)
    s = jnp.sum(p, axis=1, keepdims=True)
    att = p / s
    h = jnp.dot(att, wh_ref[...], precision=_PREC,
                preferred_element_type=jnp.float32)
    if elu:
        h = jnp.where(h > 0, h, jnp.exp(h) - 1.0)
    o_ref[...] = h


def _gat(adj, fs, fd_row, Wh, elu):
    d = Wh.shape[1]
    return pl.pallas_call(
        functools.partial(_gat_body, elu=elu),
        grid=(N // BM,),
        in_specs=[
            pl.BlockSpec((BM, N), _rows),
            pl.BlockSpec((BM, 1), _rows),
            pl.BlockSpec((1, N), _const),
            pl.BlockSpec((N, d), _const),
        ],
        out_specs=pl.BlockSpec((BM, d), _rows),
        out_shape=jax.ShapeDtypeStruct((N, d), jnp.float32),
    )(adj, fs, fd_row, Wh)


def _mix_body(h_ref, t_ref, wg_ref, as_ref, ad_ref, wh_ref, fs_ref, fd_ref):
    mix = 0.5 * h_ref[...] + 0.5 * t_ref[...]
    wh = jnp.dot(mix, wg_ref[...], precision=_PREC,
                 preferred_element_type=jnp.float32)
    wh_ref[...] = wh
    fs_ref[...] = jnp.sum(wh * as_ref[...], axis=1, keepdims=True)
    fd_ref[...] = jnp.sum(wh * ad_ref[...], axis=1, keepdims=True)


def _mix_project(h, tra, Wg, as_row, ad_row):
    e = h.shape[1]
    f = Wg.shape[1]
    return pl.pallas_call(
        _mix_body,
        grid=(N // BM,),
        in_specs=[
            pl.BlockSpec((BM, e), _rows),
            pl.BlockSpec((BM, e), _rows),
            pl.BlockSpec((e, f), _const),
            pl.BlockSpec((1, f), _const),
            pl.BlockSpec((1, f), _const),
        ],
        out_specs=[
            pl.BlockSpec((BM, f), _rows),
            pl.BlockSpec((BM, 1), _rows),
            pl.BlockSpec((BM, 1), _rows),
        ],
        out_shape=[
            jax.ShapeDtypeStruct((N, f), jnp.float32),
            jax.ShapeDtypeStruct((N, 1), jnp.float32),
            jax.ShapeDtypeStruct((N, 1), jnp.float32),
        ],
    )(h, tra, Wg, as_row, ad_row)


def _fuse_body(z_ref, ha_ref, hk_ref, hd_ref, wp1_ref, bp1_ref, wp2_ref,
               emb_ref, ba_ref, bk_ref, bd_ref):
    wp1 = wp1_ref[...]
    bp1 = bp1_ref[...]
    wp2 = wp2_ref[...]

    def score(u):
        t = jnp.tanh(jnp.dot(u, wp1, precision=_PREC,
                             preferred_element_type=jnp.float32) + bp1)
        return jnp.sum(t * wp2, axis=1, keepdims=True)

    zb = z_ref[...]
    wz = score(zb)
    embs = []
    for h_ref, b_ref in ((ha_ref, ba_ref), (hk_ref, bk_ref), (hd_ref, bd_ref)):
        hb = h_ref[...]
        wh = score(hb)
        m = jnp.maximum(wh, wz)
        p1 = jnp.exp(wh - m)
        p2 = jnp.exp(wz - m)
        s = p1 + p2
        b1 = p1 / s
        b2 = p2 / s
        b_ref[...] = jnp.concatenate([b1, b2], axis=1)
        embs.append(b1 * hb + b2 * zb)

    w1, w2, w3 = score(embs[0]), score(embs[1]), score(embs[2])
    m = jnp.maximum(jnp.maximum(w1, w2), w3)
    p1 = jnp.exp(w1 - m)
    p2 = jnp.exp(w2 - m)
    p3 = jnp.exp(w3 - m)
    s = p1 + p2 + p3
    emb_ref[...] = (p1 / s) * embs[0] + (p2 / s) * embs[1] + (p3 / s) * embs[2]


def _fuse(z, h3a, h3k, h3d, Wp1, bp1_row, wp2_row):
    nz = z.shape[1]
    return pl.pallas_call(
        _fuse_body,
        grid=(N // BM,),
        in_specs=[
            pl.BlockSpec((BM, nz), _rows),
            pl.BlockSpec((BM, nz), _rows),
            pl.BlockSpec((BM, nz), _rows),
            pl.BlockSpec((BM, nz), _rows),
            pl.BlockSpec((nz, nz), _const),
            pl.BlockSpec((1, nz), _const),
            pl.BlockSpec((1, nz), _const),
        ],
        out_specs=[
            pl.BlockSpec((BM, nz), _rows),
            pl.BlockSpec((BM, 2), _rows),
            pl.BlockSpec((BM, 2), _rows),
            pl.BlockSpec((BM, 2), _rows),
        ],
        out_shape=[
            jax.ShapeDtypeStruct((N, nz), jnp.float32),
            jax.ShapeDtypeStruct((N, 2), jnp.float32),
            jax.ShapeDtypeStruct((N, 2), jnp.float32),
            jax.ShapeDtypeStruct((N, 2), jnp.float32),
        ],
    )(z, h3a, h3k, h3d, Wp1, bp1_row, wp2_row)


def kernel(x, adj, adj_knn, adj_diff, We1, be1, We2, be2, Wz, bz, Wd1, bd1,
           Wd2, bd2, Wxb, bxb, Wg1, ag1s, ag1d, Wg2, ag2s, ag2d, Wg3, ag3s,
           ag3d, Wp1, bp1, Wp2):
    enc_h1, Wh1, f1s, f1d = _k1(x, We1, be1.reshape(1, -1), Wg1,
                                ag1s.reshape(1, -1), ag1d.reshape(1, -1))
    f1d_row = f1d.reshape(1, N)

    enc_h2, z, x_bar = _k2(enc_h1, We2, be2.reshape(1, -1), Wz,
                           bz.reshape(1, -1), Wd1, bd1.reshape(1, -1),
                           Wd2, bd2.reshape(1, -1), Wxb, bxb.reshape(1, -1))

    a2s_row = ag2s.reshape(1, -1)
    a2d_row = ag2d.reshape(1, -1)
    a3s_row = ag3s.reshape(1, -1)
    a3d_row = ag3d.reshape(1, -1)

    h3 = {}
    for name, adj_12, adj_3 in (("adj", adj, adj),
                                ("knn", adj_knn, adj),
                                ("diff", adj_diff, adj_diff)):
        h1_v = _gat(adj_12, f1s, f1d_row, Wh1, elu=True)
        Wh2, f2s, f2d = _mix_project(h1_v, enc_h1, Wg2, a2s_row, a2d_row)
        h2_v = _gat(adj_12, f2s, f2d.reshape(1, N), Wh2, elu=True)
        Wh3, f3s, f3d = _mix_project(h2_v, enc_h2, Wg3, a3s_row, a3d_row)
        h3[name] = _gat(adj_3, f3s, f3d.reshape(1, N), Wh3, elu=False)

    emb_last, b_adj, b_knn, b_diff = _fuse(
        z, h3["adj"], h3["knn"], h3["diff"], Wp1, bp1.reshape(1, -1),
        Wp2.reshape(1, -1))

    return (emb_last,
            b_adj.reshape(N, 2, 1),
            b_knn.reshape(N, 2, 1),
            b_diff.reshape(N, 2, 1),
            x_bar)


# speedup vs baseline: 2.7318x; 2.5468x over previous
import functools

import jax
import jax.numpy as jnp
from jax.experimental import pallas as pl

N = 2048
BM = 256
_PREC = jax.lax.Precision.DEFAULT


def _rows(i):
    return (i, 0)


def _const(i):
    return (0, 0)


def _k1_body(x_ref, we1_ref, be1_ref, wg1_ref, a1s_ref, a1d_ref,
             enc_ref, wh_ref, fs_ref, fd_ref):
    xb = x_ref[...]
    enc = jnp.dot(xb, we1_ref[...], precision=_PREC,
                  preferred_element_type=jnp.float32) + be1_ref[...]
    enc_ref[...] = jnp.maximum(enc, 0.0)
    wh = jnp.dot(xb, wg1_ref[...], precision=_PREC,
                 preferred_element_type=jnp.float32)
    wh_ref[...] = wh
    fs_ref[...] = jnp.sum(wh * a1s_ref[...], axis=1, keepdims=True)
    fd_ref[...] = jnp.sum(wh * a1d_ref[...], axis=1, keepdims=True)


def _k1(x_pad, We1p, be1p, Wg1p, a1s_row, a1d_row):
    dp = x_pad.shape[1]
    e1 = We1p.shape[1]
    return pl.pallas_call(
        _k1_body,
        grid=(N // BM,),
        in_specs=[
            pl.BlockSpec((BM, dp), _rows),
            pl.BlockSpec((dp, e1), _const),
            pl.BlockSpec((1, e1), _const),
            pl.BlockSpec((dp, e1), _const),
            pl.BlockSpec((1, e1), _const),
            pl.BlockSpec((1, e1), _const),
        ],
        out_specs=[
            pl.BlockSpec((BM, e1), _rows),
            pl.BlockSpec((BM, e1), _rows),
            pl.BlockSpec((BM, 1), _rows),
            pl.BlockSpec((BM, 1), _rows),
        ],
        out_shape=[
            jax.ShapeDtypeStruct((N, e1), jnp.float32),
            jax.ShapeDtypeStruct((N, e1), jnp.float32),
            jax.ShapeDtypeStruct((N, 1), jnp.float32),
            jax.ShapeDtypeStruct((N, 1), jnp.float32),
        ],
    )(x_pad, We1p, be1p, Wg1p, a1s_row, a1d_row)


def _k2_body(enc_ref, we2_ref, be2_ref, wz_ref, bz_ref, wd1_ref, bd1_ref,
             wd2_ref, bd2_ref, wxb_ref, bxb_ref, h2_ref, z_ref, xb_ref):
    h1 = enc_ref[...]
    h2 = jnp.maximum(jnp.dot(h1, we2_ref[...], precision=_PREC,
                             preferred_element_type=jnp.float32)
                     + be2_ref[...], 0.0)
    h2_ref[...] = h2
    z = jnp.dot(h2, wz_ref[...], precision=_PREC,
                preferred_element_type=jnp.float32) + bz_ref[...]
    z_ref[...] = z
    d1 = jnp.maximum(jnp.dot(z, wd1_ref[...], precision=_PREC,
                             preferred_element_type=jnp.float32)
                     + bd1_ref[...], 0.0)
    d2 = jnp.maximum(jnp.dot(d1, wd2_ref[...], precision=_PREC,
                             preferred_element_type=jnp.float32)
                     + bd2_ref[...], 0.0)
    xb_ref[...] = jnp.dot(d2, wxb_ref[...], precision=_PREC,
                          preferred_element_type=jnp.float32) + bxb_ref[...]


def _k2(enc_h1, We2p, be2, Wz, bz, Wd1, bd1, Wd2p, bd2p, Wxbp, bxbp):
    e1 = enc_h1.shape[1]
    e2 = We2p.shape[1]
    nz = Wz.shape[1]
    dp = Wxbp.shape[1]
    return pl.pallas_call(
        _k2_body,
        grid=(N // BM,),
        in_specs=[
            pl.BlockSpec((BM, e1), _rows),
            pl.BlockSpec((e1, e2), _const),
            pl.BlockSpec((1, e2), _const),
            pl.BlockSpec((e2, nz), _const),
            pl.BlockSpec((1, nz), _const),
            pl.BlockSpec((nz, e2), _const),
            pl.BlockSpec((1, e2), _const),
            pl.BlockSpec((e2, e1), _const),
            pl.BlockSpec((1, e1), _const),
            pl.BlockSpec((e1, dp), _const),
            pl.BlockSpec((1, dp), _const),
        ],
        out_specs=[
            pl.BlockSpec((BM, e2), _rows),
            pl.BlockSpec((BM, nz), _rows),
            pl.BlockSpec((BM, dp), _rows),
        ],
        out_shape=[
            jax.ShapeDtypeStruct((N, e2), jnp.float32),
            jax.ShapeDtypeStruct((N, nz), jnp.float32),
            jax.ShapeDtypeStruct((N, dp), jnp.float32),
        ],
    )(enc_h1, We2p, be2, Wz, bz, Wd1, bd1, Wd2p, bd2p, Wxbp, bxbp)


def _gat_body(adj_ref, fs_ref, fd_ref, wh_ref, o_ref, *, elu):
    e = fs_ref[...] + fd_ref[...]
    e = jnp.where(e >= 0, e, 0.2 * e)
    masked = jnp.where(adj_ref[...] > 0, e, jnp.float32(-9e15))
    m = jnp.max(masked, axis=1, keepdims=True)
    p = jnp.exp(masked - m)
    s = jnp.sum(p, axis=1, keepdims=True)
    att = p / s
    h = jnp.dot(att, wh_ref[...], precision=_PREC,
                preferred_element_type=jnp.float32)
    if elu:
        h = jnp.where(h > 0, h, jnp.exp(h) - 1.0)
    o_ref[...] = h


def _gat(adj, fs, fd_row, Wh, elu):
    d = Wh.shape[1]
    return pl.pallas_call(
        functools.partial(_gat_body, elu=elu),
        grid=(N // BM,),
        in_specs=[
            pl.BlockSpec((BM, N), _rows),
            pl.BlockSpec((BM, 1), _rows),
            pl.BlockSpec((1, N), _const),
            pl.BlockSpec((N, d), _const),
        ],
        out_specs=pl.BlockSpec((BM, d), _rows),
        out_shape=jax.ShapeDtypeStruct((N, d), jnp.float32),
    )(adj, fs, fd_row, Wh)


def _mix_body(h_ref, t_ref, wg_ref, as_ref, ad_ref, wh_ref, fs_ref, fd_ref):
    mix = 0.5 * h_ref[...] + 0.5 * t_ref[...]
    wh = jnp.dot(mix, wg_ref[...], precision=_PREC,
                 preferred_element_type=jnp.float32)
    wh_ref[...] = wh
    fs_ref[...] = jnp.sum(wh * as_ref[...], axis=1, keepdims=True)
    fd_ref[...] = jnp.sum(wh * ad_ref[...], axis=1, keepdims=True)


def _mix_project(h, tra, Wg, as_row, ad_row):
    e = h.shape[1]
    f = Wg.shape[1]
    return pl.pallas_call(
        _mix_body,
        grid=(N // BM,),
        in_specs=[
            pl.BlockSpec((BM, e), _rows),
            pl.BlockSpec((BM, e), _rows),
            pl.BlockSpec((e, f), _const),
            pl.BlockSpec((1, f), _const),
            pl.BlockSpec((1, f), _const),
        ],
        out_specs=[
            pl.BlockSpec((BM, f), _rows),
            pl.BlockSpec((BM, 1), _rows),
            pl.BlockSpec((BM, 1), _rows),
        ],
        out_shape=[
            jax.ShapeDtypeStruct((N, f), jnp.float32),
            jax.ShapeDtypeStruct((N, 1), jnp.float32),
            jax.ShapeDtypeStruct((N, 1), jnp.float32),
        ],
    )(h, tra, Wg, as_row, ad_row)


def _fuse_body(z_ref, ha_ref, hk_ref, hd_ref, wp1_ref, bp1_ref, wp2_ref,
               emb_ref, ba_ref, bk_ref, bd_ref):
    wp1 = wp1_ref[...]
    bp1 = bp1_ref[...]
    wp2 = wp2_ref[...]

    def score(u):
        t = jnp.tanh(jnp.dot(u, wp1, precision=_PREC,
                             preferred_element_type=jnp.float32) + bp1)
        return jnp.sum(t * wp2, axis=1, keepdims=True)

    zb = z_ref[...]
    wz = score(zb)
    embs = []
    for h_ref, b_ref in ((ha_ref, ba_ref), (hk_ref, bk_ref), (hd_ref, bd_ref)):
        hb = h_ref[...]
        wh = score(hb)
        m = jnp.maximum(wh, wz)
        p1 = jnp.exp(wh - m)
        p2 = jnp.exp(wz - m)
        s = p1 + p2
        b1 = p1 / s
        b2 = p2 / s
        b_ref[...] = jnp.concatenate([b1, b2], axis=1)
        embs.append(b1 * hb + b2 * zb)

    w1, w2, w3 = score(embs[0]), score(embs[1]), score(embs[2])
    m = jnp.maximum(jnp.maximum(w1, w2), w3)
    p1 = jnp.exp(w1 - m)
    p2 = jnp.exp(w2 - m)
    p3 = jnp.exp(w3 - m)
    s = p1 + p2 + p3
    emb_ref[...] = (p1 / s) * embs[0] + (p2 / s) * embs[1] + (p3 / s) * embs[2]


def _fuse(z, h3a, h3k, h3d, Wp1, bp1_row, wp2_row):
    nz = z.shape[1]
    return pl.pallas_call(
        _fuse_body,
        grid=(N // BM,),
        in_specs=[
            pl.BlockSpec((BM, nz), _rows),
            pl.BlockSpec((BM, nz), _rows),
            pl.BlockSpec((BM, nz), _rows),
            pl.BlockSpec((BM, nz), _rows),
            pl.BlockSpec((nz, nz), _const),
            pl.BlockSpec((1, nz), _const),
            pl.BlockSpec((1, nz), _const),
        ],
        out_specs=[
            pl.BlockSpec((BM, nz), _rows),
            pl.BlockSpec((BM, 2), _rows),
            pl.BlockSpec((BM, 2), _rows),
            pl.BlockSpec((BM, 2), _rows),
        ],
        out_shape=[
            jax.ShapeDtypeStruct((N, nz), jnp.float32),
            jax.ShapeDtypeStruct((N, 2), jnp.float32),
            jax.ShapeDtypeStruct((N, 2), jnp.float32),
            jax.ShapeDtypeStruct((N, 2), jnp.float32),
        ],
    )(z, h3a, h3k, h3d, Wp1, bp1_row, wp2_row)


def kernel(x, adj, adj_knn, adj_diff, We1, be1, We2, be2, Wz, bz, Wd1, bd1,
           Wd2, bd2, Wxb, bxb, Wg1, ag1s, ag1d, Wg2, ag2s, ag2d, Wg3, ag3s,
           ag3d, Wp1, bp1, Wp2):
    enc_h1, Wh1, f1s, f1d = _k1(x, We1, be1.reshape(1, -1), Wg1,
                                ag1s.reshape(1, -1), ag1d.reshape(1, -1))
    f1d_row = f1d.reshape(1, N)

    enc_h2, z, x_bar = _k2(enc_h1, We2, be2.reshape(1, -1), Wz,
                           bz.reshape(1, -1), Wd1, bd1.reshape(1, -1),
                           Wd2, bd2.reshape(1, -1), Wxb, bxb.reshape(1, -1))

    a2s_row = ag2s.reshape(1, -1)
    a2d_row = ag2d.reshape(1, -1)
    a3s_row = ag3s.reshape(1, -1)
    a3d_row = ag3d.reshape(1, -1)

    h3 = {}
    for name, adj_12, adj_3 in (("adj", adj, adj),
                                ("knn", adj_knn, adj),
                                ("diff", adj_diff, adj_diff)):
        h1_v = _gat(adj_12, f1s, f1d_row, Wh1, elu=True)
        Wh2, f2s, f2d = _mix_project(h1_v, enc_h1, Wg2, a2s_row, a2d_row)
        h2_v = _gat(adj_12, f2s, f2d.reshape(1, N), Wh2, elu=True)
        Wh3, f3s, f3d = _mix_project(h2_v, enc_h2, Wg3, a3s_row, a3d_row)
        h3[name] = _gat(adj_3, f3s, f3d.reshape(1, N), Wh3, elu=False)

    emb_last, b_adj, b_knn, b_diff = _fuse(
        z, h3["adj"], h3["knn"], h3["diff"], Wp1, bp1.reshape(1, -1),
        Wp2.reshape(1, -1))

    return (emb_last,
            b_adj.reshape(N, 2, 1),
            b_knn.reshape(N, 2, 1),
            b_diff.reshape(N, 2, 1),
            x_bar)


# speedup vs baseline: 3.6307x; 1.3290x over previous
import functools

import jax
import jax.numpy as jnp
from jax.experimental import pallas as pl

N = 2048
BM = 256
_PREC = jax.lax.Precision.DEFAULT


def _rows(i):
    return (i, 0)


def _const(i):
    return (0, 0)


def _dot(a, b):
    return jnp.dot(a, b, precision=_PREC, preferred_element_type=jnp.float32)


def _masked_att_agg(adj, e, wh):
    masked = jnp.where(adj > 0, e, jnp.float32(-9e15))
    m = jnp.max(masked, axis=1, keepdims=True)
    p = jnp.exp(masked - m)
    att = p / jnp.sum(p, axis=1, keepdims=True)
    return _dot(att, wh)


def _leaky(x):
    return jnp.where(x >= 0, x, 0.2 * x)


def _elu(x):
    return jnp.where(x > 0, x, jnp.exp(x) - 1.0)


def _k1_body(x_ref, we1_ref, be1_ref, wg1_ref, a1s_ref, a1d_ref,
             enc_ref, wh_ref, fs_ref, fd_ref):
    xb = x_ref[...]
    enc_ref[...] = jnp.maximum(_dot(xb, we1_ref[...]) + be1_ref[...], 0.0)
    wh = _dot(xb, wg1_ref[...])
    wh_ref[...] = wh
    fs_ref[...] = jnp.sum(wh * a1s_ref[...], axis=1, keepdims=True)
    fd_ref[...] = jnp.sum(wh * a1d_ref[...], axis=1, keepdims=True)


def _k1(x, We1, be1, Wg1, a1s_row, a1d_row):
    d_in = x.shape[1]
    e1 = We1.shape[1]
    return pl.pallas_call(
        _k1_body,
        grid=(N // BM,),
        in_specs=[
            pl.BlockSpec((BM, d_in), _rows),
            pl.BlockSpec((d_in, e1), _const),
            pl.BlockSpec((1, e1), _const),
            pl.BlockSpec((d_in, e1), _const),
            pl.BlockSpec((1, e1), _const),
            pl.BlockSpec((1, e1), _const),
        ],
        out_specs=[
            pl.BlockSpec((BM, e1), _rows),
            pl.BlockSpec((BM, e1), _rows),
            pl.BlockSpec((BM, 1), _rows),
            pl.BlockSpec((BM, 1), _rows),
        ],
        out_shape=[
            jax.ShapeDtypeStruct((N, e1), jnp.float32),
            jax.ShapeDtypeStruct((N, e1), jnp.float32),
            jax.ShapeDtypeStruct((N, 1), jnp.float32),
            jax.ShapeDtypeStruct((N, 1), jnp.float32),
        ],
    )(x, We1, be1, Wg1, a1s_row, a1d_row)


def _k2_body(enc_ref, we2_ref, be2_ref, wz_ref, bz_ref, wd1_ref, bd1_ref,
             wd2_ref, bd2_ref, wxb_ref, bxb_ref, h2_ref, z_ref, xb_ref):
    h1 = enc_ref[...]
    h2 = jnp.maximum(_dot(h1, we2_ref[...]) + be2_ref[...], 0.0)
    h2_ref[...] = h2
    z = _dot(h2, wz_ref[...]) + bz_ref[...]
    z_ref[...] = z
    d1 = jnp.maximum(_dot(z, wd1_ref[...]) + bd1_ref[...], 0.0)
    d2 = jnp.maximum(_dot(d1, wd2_ref[...]) + bd2_ref[...], 0.0)
    xb_ref[...] = _dot(d2, wxb_ref[...]) + bxb_ref[...]


def _k2(enc_h1, We2, be2, Wz, bz, Wd1, bd1, Wd2, bd2, Wxb, bxb):
    e1 = enc_h1.shape[1]
    e2 = We2.shape[1]
    nz = Wz.shape[1]
    d_in = Wxb.shape[1]
    return pl.pallas_call(
        _k2_body,
        grid=(N // BM,),
        in_specs=[
            pl.BlockSpec((BM, e1), _rows),
            pl.BlockSpec((e1, e2), _const),
            pl.BlockSpec((1, e2), _const),
            pl.BlockSpec((e2, nz), _const),
            pl.BlockSpec((1, nz), _const),
            pl.BlockSpec((nz, e2), _const),
            pl.BlockSpec((1, e2), _const),
            pl.BlockSpec((e2, e1), _const),
            pl.BlockSpec((1, e1), _const),
            pl.BlockSpec((e1, d_in), _const),
            pl.BlockSpec((1, d_in), _const),
        ],
        out_specs=[
            pl.BlockSpec((BM, e2), _rows),
            pl.BlockSpec((BM, nz), _rows),
            pl.BlockSpec((BM, d_in), _rows),
        ],
        out_shape=[
            jax.ShapeDtypeStruct((N, e2), jnp.float32),
            jax.ShapeDtypeStruct((N, nz), jnp.float32),
            jax.ShapeDtypeStruct((N, d_in), jnp.float32),
        ],
    )(enc_h1, We2, be2, Wz, bz, Wd1, bd1, Wd2, bd2, Wxb, bxb)


def _g1m2_body(adj1_ref, adj2_ref, adj3_ref, fs_ref, fd_ref, wh1_ref,
               enc_ref, wg2_ref, a2s_ref, a2d_ref,
               wh2a_ref, wh2k_ref, wh2d_ref,
               fsa_ref, fda_ref, fsk_ref, fdk_ref, fsd_ref, fdd_ref):
    e = _leaky(fs_ref[...] + fd_ref[...])
    wh1 = wh1_ref[...]
    mixb = 0.5 * enc_ref[...]
    wg2 = wg2_ref[...]
    a2s = a2s_ref[...]
    a2d = a2d_ref[...]
    for adj_ref, wh2_ref, f2s_ref, f2d_ref in (
            (adj1_ref, wh2a_ref, fsa_ref, fda_ref),
            (adj2_ref, wh2k_ref, fsk_ref, fdk_ref),
            (adj3_ref, wh2d_ref, fsd_ref, fdd_ref)):
        h1 = _elu(_masked_att_agg(adj_ref[...], e, wh1))
        wh2 = _dot(0.5 * h1 + mixb, wg2)
        wh2_ref[...] = wh2
        f2s_ref[...] = jnp.sum(wh2 * a2s, axis=1, keepdims=True)
        f2d_ref[...] = jnp.sum(wh2 * a2d, axis=1, keepdims=True)


def _g1m2(adj, adj_knn, adj_diff, f1s, f1d_row, Wh1, enc_h1, Wg2,
          a2s_row, a2d_row):
    e1 = Wh1.shape[1]
    e2 = Wg2.shape[1]
    fcol = [
        jax.ShapeDtypeStruct((N, 1), jnp.float32),
        jax.ShapeDtypeStruct((N, 1), jnp.float32),
    ]
    return pl.pallas_call(
        _g1m2_body,
        grid=(N // BM,),
        in_specs=[
            pl.BlockSpec((BM, N), _rows),
            pl.BlockSpec((BM, N), _rows),
            pl.BlockSpec((BM, N), _rows),
            pl.BlockSpec((BM, 1), _rows),
            pl.BlockSpec((1, N), _const),
            pl.BlockSpec((N, e1), _const),
            pl.BlockSpec((BM, e1), _rows),
            pl.BlockSpec((e1, e2), _const),
            pl.BlockSpec((1, e2), _const),
            pl.BlockSpec((1, e2), _const),
        ],
        out_specs=[
            pl.BlockSpec((BM, e2), _rows),
            pl.BlockSpec((BM, e2), _rows),
            pl.BlockSpec((BM, e2), _rows),
            pl.BlockSpec((BM, 1), _rows),
            pl.BlockSpec((BM, 1), _rows),
            pl.BlockSpec((BM, 1), _rows),
            pl.BlockSpec((BM, 1), _rows),
            pl.BlockSpec((BM, 1), _rows),
            pl.BlockSpec((BM, 1), _rows),
        ],
        out_shape=[
            jax.ShapeDtypeStruct((N, e2), jnp.float32),
            jax.ShapeDtypeStruct((N, e2), jnp.float32),
            jax.ShapeDtypeStruct((N, e2), jnp.float32),
        ] + fcol + fcol + fcol,
    )(adj, adj_knn, adj_diff, f1s, f1d_row, Wh1, enc_h1, Wg2,
      a2s_row, a2d_row)


def _g2m3_body(adj1_ref, adj2_ref, adj3_ref,
               fsa_ref, fda_ref, fsk_ref, fdk_ref, fsd_ref, fdd_ref,
               wha_ref, whk_ref, whd_ref, enc2_ref, wg3_ref,
               a3s_ref, a3d_ref,
               wh3a_ref, wh3k_ref, wh3d_ref,
               osa_ref, oda_ref, osk_ref, odk_ref, osd_ref, odd_ref):
    mixb = 0.5 * enc2_ref[...]
    wg3 = wg3_ref[...]
    a3s = a3s_ref[...]
    a3d = a3d_ref[...]
    for adj_ref, fs_ref, fd_ref, wh_ref, wh3_ref, os_ref, od_ref in (
            (adj1_ref, fsa_ref, fda_ref, wha_ref, wh3a_ref, osa_ref, oda_ref),
            (adj2_ref, fsk_ref, fdk_ref, whk_ref, wh3k_ref, osk_ref, odk_ref),
            (adj3_ref, fsd_ref, fdd_ref, whd_ref, wh3d_ref, osd_ref, odd_ref)):
        e = _leaky(fs_ref[...] + fd_ref[...])
        h2 = _elu(_masked_att_agg(adj_ref[...], e, wh_ref[...]))
        wh3 = _dot(0.5 * h2 + mixb, wg3)
        wh3_ref[...] = wh3
        os_ref[...] = jnp.sum(wh3 * a3s, axis=1, keepdims=True)
        od_ref[...] = jnp.sum(wh3 * a3d, axis=1, keepdims=True)


def _g2m3(adj, adj_knn, adj_diff, f2, Wh2, enc_h2, Wg3, a3s_row, a3d_row):
    e2 = Wg3.shape[0]
    nz = Wg3.shape[1]
    fcol = [
        jax.ShapeDtypeStruct((N, 1), jnp.float32),
        jax.ShapeDtypeStruct((N, 1), jnp.float32),
    ]
    f_specs = []
    f_args = []
    for fs, fd in f2:
        f_specs += [pl.BlockSpec((BM, 1), _rows), pl.BlockSpec((1, N), _const)]
        f_args += [fs, fd.reshape(1, N)]
    return pl.pallas_call(
        _g2m3_body,
        grid=(N // BM,),
        in_specs=[
            pl.BlockSpec((BM, N), _rows),
            pl.BlockSpec((BM, N), _rows),
            pl.BlockSpec((BM, N), _rows),
        ] + f_specs + [
            pl.BlockSpec((N, e2), _const),
            pl.BlockSpec((N, e2), _const),
            pl.BlockSpec((N, e2), _const),
            pl.BlockSpec((BM, e2), _rows),
            pl.BlockSpec((e2, nz), _const),
            pl.BlockSpec((1, nz), _const),
            pl.BlockSpec((1, nz), _const),
        ],
        out_specs=[
            pl.BlockSpec((BM, nz), _rows),
            pl.BlockSpec((BM, nz), _rows),
            pl.BlockSpec((BM, nz), _rows),
            pl.BlockSpec((BM, 1), _rows),
            pl.BlockSpec((BM, 1), _rows),
            pl.BlockSpec((BM, 1), _rows),
            pl.BlockSpec((BM, 1), _rows),
            pl.BlockSpec((BM, 1), _rows),
            pl.BlockSpec((BM, 1), _rows),
        ],
        out_shape=[
            jax.ShapeDtypeStruct((N, nz), jnp.float32),
            jax.ShapeDtypeStruct((N, nz), jnp.float32),
            jax.ShapeDtypeStruct((N, nz), jnp.float32),
        ] + fcol + fcol + fcol,
    )(adj, adj_knn, adj_diff, *f_args, Wh2[0], Wh2[1], Wh2[2], enc_h2,
      Wg3, a3s_row, a3d_row)


def _g3k5_body(adj_ref, adjd_ref,
               fsa_ref, fda_ref, fsk_ref, fdk_ref, fsd_ref, fdd_ref,
               wha_ref, whk_ref, whd_ref, z_ref,
               wp1_ref, bp1_ref, wp2_ref,
               emb_ref, ba_ref, bk_ref, bd_ref):
    wp1 = wp1_ref[...]
    bp1 = bp1_ref[...]
    wp2 = wp2_ref[...]

    def score(u):
        t = jnp.tanh(_dot(u, wp1) + bp1)
        return jnp.sum(t * wp2, axis=1, keepdims=True)

    zb = z_ref[...]
    wz = score(zb)
    embs = []
    for a_ref, fs_ref, fd_ref, wh_ref, b_ref in (
            (adj_ref, fsa_ref, fda_ref, wha_ref, ba_ref),
            (adj_ref, fsk_ref, fdk_ref, whk_ref, bk_ref),
            (adjd_ref, fsd_ref, fdd_ref, whd_ref, bd_ref)):
        e = _leaky(fs_ref[...] + fd_ref[...])
        h3 = _masked_att_agg(a_ref[...], e, wh_ref[...])
        wh = score(h3)
        m = jnp.maximum(wh, wz)
        p1 = jnp.exp(wh - m)
        p2 = jnp.exp(wz - m)
        s = p1 + p2
        b1 = p1 / s
        b2 = p2 / s
        b_ref[...] = jnp.concatenate([b1, b2], axis=1)
        embs.append(b1 * h3 + b2 * zb)

    w1, w2, w3 = score(embs[0]), score(embs[1]), score(embs[2])
    m = jnp.maximum(jnp.maximum(w1, w2), w3)
    p1 = jnp.exp(w1 - m)
    p2 = jnp.exp(w2 - m)
    p3 = jnp.exp(w3 - m)
    s = p1 + p2 + p3
    emb_ref[...] = (p1 / s) * embs[0] + (p2 / s) * embs[1] + (p3 / s) * embs[2]


def _g3k5(adj, adj_diff, f3, Wh3, z, Wp1, bp1_row, wp2_row):
    nz = z.shape[1]
    f_specs = []
    f_args = []
    for fs, fd in f3:
        f_specs += [pl.BlockSpec((BM, 1), _rows), pl.BlockSpec((1, N), _const)]
        f_args += [fs, fd.reshape(1, N)]
    return pl.pallas_call(
        _g3k5_body,
        grid=(N // BM,),
        in_specs=[
            pl.BlockSpec((BM, N), _rows),
            pl.BlockSpec((BM, N), _rows),
        ] + f_specs + [
            pl.BlockSpec((N, nz), _const),
            pl.BlockSpec((N, nz), _const),
            pl.BlockSpec((N, nz), _const),
            pl.BlockSpec((BM, nz), _rows),
            pl.BlockSpec((nz, nz), _const),
            pl.BlockSpec((1, nz), _const),
            pl.BlockSpec((1, nz), _const),
        ],
        out_specs=[
            pl.BlockSpec((BM, nz), _rows),
            pl.BlockSpec((BM, 2), _rows),
            pl.BlockSpec((BM, 2), _rows),
            pl.BlockSpec((BM, 2), _rows),
        ],
        out_shape=[
            jax.ShapeDtypeStruct((N, nz), jnp.float32),
            jax.ShapeDtypeStruct((N, 2), jnp.float32),
            jax.ShapeDtypeStruct((N, 2), jnp.float32),
            jax.ShapeDtypeStruct((N, 2), jnp.float32),
        ],
    )(adj, adj_diff, *f_args, Wh3[0], Wh3[1], Wh3[2], z, Wp1,
      bp1_row, wp2_row)


def kernel(x, adj, adj_knn, adj_diff, We1, be1, We2, be2, Wz, bz, Wd1, bd1,
           Wd2, bd2, Wxb, bxb, Wg1, ag1s, ag1d, Wg2, ag2s, ag2d, Wg3, ag3s,
           ag3d, Wp1, bp1, Wp2):
    enc_h1, Wh1, f1s, f1d = _k1(x, We1, be1.reshape(1, -1), Wg1,
                                ag1s.reshape(1, -1), ag1d.reshape(1, -1))

    enc_h2, z, x_bar = _k2(enc_h1, We2, be2.reshape(1, -1), Wz,
                           bz.reshape(1, -1), Wd1, bd1.reshape(1, -1),
                           Wd2, bd2.reshape(1, -1), Wxb, bxb.reshape(1, -1))

    wh2a, wh2k, wh2d, fsa, fda, fsk, fdk, fsd, fdd = _g1m2(
        adj, adj_knn, adj_diff, f1s, f1d.reshape(1, N), Wh1, enc_h1, Wg2,
        ag2s.reshape(1, -1), ag2d.reshape(1, -1))

    wh3a, wh3k, wh3d, osa, oda, osk, odk, osd, odd = _g2m3(
        adj, adj_knn, adj_diff,
        ((fsa, fda), (fsk, fdk), (fsd, fdd)),
        (wh2a, wh2k, wh2d), enc_h2, Wg3,
        ag3s.reshape(1, -1), ag3d.reshape(1, -1))

    emb_last, b_adj, b_knn, b_diff = _g3k5(
        adj, adj_diff,
        ((osa, oda), (osk, odk), (osd, odd)),
        (wh3a, wh3k, wh3d), z, Wp1, bp1.reshape(1, -1), Wp2.reshape(1, -1))

    return (emb_last,
            b_adj.reshape(N, 2, 1),
            b_knn.reshape(N, 2, 1),
            b_diff.reshape(N, 2, 1),
            x_bar)


# speedup vs baseline: 3.7362x; 1.0291x over previous
import functools

import jax
import jax.numpy as jnp
from jax.experimental import pallas as pl

N = 2048
BM = 256
_PREC = jax.lax.Precision.DEFAULT


def _rows(i):
    return (i, 0)


def _const(i):
    return (0, 0)


def _dot(a, b):
    return jnp.dot(a, b, precision=_PREC, preferred_element_type=jnp.float32)


def _masked_exp(adj, e):
    masked = jnp.where(adj > 0, e, jnp.float32(-9e15))
    m = jnp.max(masked, axis=1, keepdims=True)
    p = jnp.exp(masked - m)
    return p, jnp.sum(p, axis=1, keepdims=True)


def _masked_att_agg(adj, e, wh):
    p, s = _masked_exp(adj, e)
    return _dot(p, wh) / s


def _leaky(x):
    return jnp.where(x >= 0, x, 0.2 * x)


def _elu(x):
    return jnp.where(x > 0, x, jnp.exp(x) - 1.0)


def _k1_body(x_ref, we1_ref, be1_ref, wg1_ref, a1s_ref, a1d_ref,
             enc_ref, wh_ref, fs_ref, fd_ref):
    xb = x_ref[...]
    enc = jnp.maximum(_dot(xb, we1_ref[...]) + be1_ref[...], 0.0)
    enc_ref[...] = enc.astype(jnp.bfloat16)
    wh = _dot(xb, wg1_ref[...])
    wh_ref[...] = wh.astype(jnp.bfloat16)
    fs_ref[...] = jnp.sum(wh * a1s_ref[...], axis=1, keepdims=True)
    fd_ref[...] = jnp.sum(wh * a1d_ref[...], axis=1, keepdims=True)


def _k1(x, We1, be1, Wg1, a1s_row, a1d_row):
    d_in = x.shape[1]
    e1 = We1.shape[1]
    return pl.pallas_call(
        _k1_body,
        grid=(N // BM,),
        in_specs=[
            pl.BlockSpec((BM, d_in), _rows),
            pl.BlockSpec((d_in, e1), _const),
            pl.BlockSpec((1, e1), _const),
            pl.BlockSpec((d_in, e1), _const),
            pl.BlockSpec((1, e1), _const),
            pl.BlockSpec((1, e1), _const),
        ],
        out_specs=[
            pl.BlockSpec((BM, e1), _rows),
            pl.BlockSpec((BM, e1), _rows),
            pl.BlockSpec((BM, 1), _rows),
            pl.BlockSpec((BM, 1), _rows),
        ],
        out_shape=[
            jax.ShapeDtypeStruct((N, e1), jnp.bfloat16),
            jax.ShapeDtypeStruct((N, e1), jnp.bfloat16),
            jax.ShapeDtypeStruct((N, 1), jnp.float32),
            jax.ShapeDtypeStruct((N, 1), jnp.float32),
        ],
    )(x, We1, be1, Wg1, a1s_row, a1d_row)


def _k2_body(enc_ref, we2_ref, be2_ref, wz_ref, bz_ref, wd1_ref, bd1_ref,
             wd2_ref, bd2_ref, wxb_ref, bxb_ref, h2_ref, z_ref, xb_ref):
    h1 = enc_ref[...]
    h2 = jnp.maximum(_dot(h1, we2_ref[...]) + be2_ref[...], 0.0)
    h2_ref[...] = h2.astype(jnp.bfloat16)
    z = _dot(h2, wz_ref[...]) + bz_ref[...]
    z_ref[...] = z
    d1 = jnp.maximum(_dot(z, wd1_ref[...]) + bd1_ref[...], 0.0)
    d2 = jnp.maximum(_dot(d1, wd2_ref[...]) + bd2_ref[...], 0.0)
    xb_ref[...] = _dot(d2, wxb_ref[...]) + bxb_ref[...]


def _k2(enc_h1, We2, be2, Wz, bz, Wd1, bd1, Wd2, bd2, Wxb, bxb):
    e1 = enc_h1.shape[1]
    e2 = We2.shape[1]
    nz = Wz.shape[1]
    d_in = Wxb.shape[1]
    return pl.pallas_call(
        _k2_body,
        grid=(N // BM,),
        in_specs=[
            pl.BlockSpec((BM, e1), _rows),
            pl.BlockSpec((e1, e2), _const),
            pl.BlockSpec((1, e2), _const),
            pl.BlockSpec((e2, nz), _const),
            pl.BlockSpec((1, nz), _const),
            pl.BlockSpec((nz, e2), _const),
            pl.BlockSpec((1, e2), _const),
            pl.BlockSpec((e2, e1), _const),
            pl.BlockSpec((1, e1), _const),
            pl.BlockSpec((e1, d_in), _const),
            pl.BlockSpec((1, d_in), _const),
        ],
        out_specs=[
            pl.BlockSpec((BM, e2), _rows),
            pl.BlockSpec((BM, nz), _rows),
            pl.BlockSpec((BM, d_in), _rows),
        ],
        out_shape=[
            jax.ShapeDtypeStruct((N, e2), jnp.bfloat16),
            jax.ShapeDtypeStruct((N, nz), jnp.float32),
            jax.ShapeDtypeStruct((N, d_in), jnp.float32),
        ],
    )(enc_h1, We2, be2, Wz, bz, Wd1, bd1, Wd2, bd2, Wxb, bxb)


def _g1m2_body(adj1_ref, adj2_ref, adj3_ref, fs_ref, fd_ref, wh1_ref,
               enc_ref, wg2_ref, a2s_ref, a2d_ref,
               wh2a_ref, wh2k_ref, wh2d_ref,
               fsa_ref, fda_ref, fsk_ref, fdk_ref, fsd_ref, fdd_ref):
    e = _leaky(fs_ref[...] + fd_ref[...])
    ps = []
    ss = []
    for adj_ref in (adj1_ref, adj2_ref, adj3_ref):
        p, sm = _masked_exp(adj_ref[...], e)
        ps.append(p)
        ss.append(sm)
    h_all = _dot(jnp.concatenate(ps, axis=0), wh1_ref[...])
    mixb = 0.5 * enc_ref[...]
    mixes = [0.5 * _elu(h_all[i * BM:(i + 1) * BM] / ss[i]) + mixb
             for i in range(3)]
    wh2_all = _dot(jnp.concatenate(mixes, axis=0), wg2_ref[...])
    a2s = a2s_ref[...]
    a2d = a2d_ref[...]
    for i, (wh2_ref, f2s_ref, f2d_ref) in enumerate(
            ((wh2a_ref, fsa_ref, fda_ref),
             (wh2k_ref, fsk_ref, fdk_ref),
             (wh2d_ref, fsd_ref, fdd_ref))):
        wh2 = wh2_all[i * BM:(i + 1) * BM]
        wh2_ref[...] = wh2.astype(jnp.bfloat16)
        f2s_ref[...] = jnp.sum(wh2 * a2s, axis=1, keepdims=True)
        f2d_ref[...] = jnp.sum(wh2 * a2d, axis=1, keepdims=True)


def _g1m2(adj, adj_knn, adj_diff, f1s, f1d_row, Wh1, enc_h1, Wg2,
          a2s_row, a2d_row):
    e1 = Wh1.shape[1]
    e2 = Wg2.shape[1]
    fcol = [
        jax.ShapeDtypeStruct((N, 1), jnp.float32),
        jax.ShapeDtypeStruct((N, 1), jnp.float32),
    ]
    return pl.pallas_call(
        _g1m2_body,
        grid=(N // BM,),
        in_specs=[
            pl.BlockSpec((BM, N), _rows),
            pl.BlockSpec((BM, N), _rows),
            pl.BlockSpec((BM, N), _rows),
            pl.BlockSpec((BM, 1), _rows),
            pl.BlockSpec((1, N), _const),
            pl.BlockSpec((N, e1), _const),
            pl.BlockSpec((BM, e1), _rows),
            pl.BlockSpec((e1, e2), _const),
            pl.BlockSpec((1, e2), _const),
            pl.BlockSpec((1, e2), _const),
        ],
        out_specs=[
            pl.BlockSpec((BM, e2), _rows),
            pl.BlockSpec((BM, e2), _rows),
            pl.BlockSpec((BM, e2), _rows),
            pl.BlockSpec((BM, 1), _rows),
            pl.BlockSpec((BM, 1), _rows),
            pl.BlockSpec((BM, 1), _rows),
            pl.BlockSpec((BM, 1), _rows),
            pl.BlockSpec((BM, 1), _rows),
            pl.BlockSpec((BM, 1), _rows),
        ],
        out_shape=[
            jax.ShapeDtypeStruct((N, e2), jnp.bfloat16),
            jax.ShapeDtypeStruct((N, e2), jnp.bfloat16),
            jax.ShapeDtypeStruct((N, e2), jnp.bfloat16),
        ] + fcol + fcol + fcol,
    )(adj, adj_knn, adj_diff, f1s, f1d_row, Wh1, enc_h1, Wg2,
      a2s_row, a2d_row)


def _g2m3_body(adj1_ref, adj2_ref, adj3_ref,
               fsa_ref, fda_ref, fsk_ref, fdk_ref, fsd_ref, fdd_ref,
               wha_ref, whk_ref, whd_ref, enc2_ref, wg3_ref,
               a3s_ref, a3d_ref,
               wh3a_ref, wh3k_ref, wh3d_ref,
               osa_ref, oda_ref, osk_ref, odk_ref, osd_ref, odd_ref):
    mixb = 0.5 * enc2_ref[...]
    wg3 = wg3_ref[...]
    a3s = a3s_ref[...]
    a3d = a3d_ref[...]
    for adj_ref, fs_ref, fd_ref, wh_ref, wh3_ref, os_ref, od_ref in (
            (adj1_ref, fsa_ref, fda_ref, wha_ref, wh3a_ref, osa_ref, oda_ref),
            (adj2_ref, fsk_ref, fdk_ref, whk_ref, wh3k_ref, osk_ref, odk_ref),
            (adj3_ref, fsd_ref, fdd_ref, whd_ref, wh3d_ref, osd_ref, odd_ref)):
        e = _leaky(fs_ref[...] + fd_ref[...])
        h2 = _elu(_masked_att_agg(adj_ref[...], e, wh_ref[...]))
        wh3 = _dot(0.5 * h2 + mixb, wg3)
        wh3_ref[...] = wh3.astype(jnp.bfloat16)
        os_ref[...] = jnp.sum(wh3 * a3s, axis=1, keepdims=True)
        od_ref[...] = jnp.sum(wh3 * a3d, axis=1, keepdims=True)


def _g2m3(adj, adj_knn, adj_diff, f2, Wh2, enc_h2, Wg3, a3s_row, a3d_row):
    e2 = Wg3.shape[0]
    nz = Wg3.shape[1]
    fcol = [
        jax.ShapeDtypeStruct((N, 1), jnp.float32),
        jax.ShapeDtypeStruct((N, 1), jnp.float32),
    ]
    f_specs = []
    f_args = []
    for fs, fd in f2:
        f_specs += [pl.BlockSpec((BM, 1), _rows), pl.BlockSpec((1, N), _const)]
        f_args += [fs, fd.reshape(1, N)]
    return pl.pallas_call(
        _g2m3_body,
        grid=(N // BM,),
        in_specs=[
            pl.BlockSpec((BM, N), _rows),
            pl.BlockSpec((BM, N), _rows),
            pl.BlockSpec((BM, N), _rows),
        ] + f_specs + [
            pl.BlockSpec((N, e2), _const),
            pl.BlockSpec((N, e2), _const),
            pl.BlockSpec((N, e2), _const),
            pl.BlockSpec((BM, e2), _rows),
            pl.BlockSpec((e2, nz), _const),
            pl.BlockSpec((1, nz), _const),
            pl.BlockSpec((1, nz), _const),
        ],
        out_specs=[
            pl.BlockSpec((BM, nz), _rows),
            pl.BlockSpec((BM, nz), _rows),
            pl.BlockSpec((BM, nz), _rows),
            pl.BlockSpec((BM, 1), _rows),
            pl.BlockSpec((BM, 1), _rows),
            pl.BlockSpec((BM, 1), _rows),
            pl.BlockSpec((BM, 1), _rows),
            pl.BlockSpec((BM, 1), _rows),
            pl.BlockSpec((BM, 1), _rows),
        ],
        out_shape=[
            jax.ShapeDtypeStruct((N, nz), jnp.bfloat16),
            jax.ShapeDtypeStruct((N, nz), jnp.bfloat16),
            jax.ShapeDtypeStruct((N, nz), jnp.bfloat16),
        ] + fcol + fcol + fcol,
    )(adj, adj_knn, adj_diff, *f_args, Wh2[0], Wh2[1], Wh2[2], enc_h2,
      Wg3, a3s_row, a3d_row)


def _g3k5_body(adj_ref, adjd_ref,
               fsa_ref, fda_ref, fsk_ref, fdk_ref, fsd_ref, fdd_ref,
               wha_ref, whk_ref, whd_ref, z_ref,
               wp1_ref, bp1_ref, wp2_ref,
               emb_ref, ba_ref, bk_ref, bd_ref):
    wp1 = wp1_ref[...]
    bp1 = bp1_ref[...]
    wp2 = wp2_ref[...]

    def score(u):
        t = jnp.tanh(_dot(u, wp1) + bp1)
        return jnp.sum(t * wp2, axis=1, keepdims=True)

    zb = z_ref[...]
    wz = score(zb)
    embs = []
    for a_ref, fs_ref, fd_ref, wh_ref, b_ref in (
            (adj_ref, fsa_ref, fda_ref, wha_ref, ba_ref),
            (adj_ref, fsk_ref, fdk_ref, whk_ref, bk_ref),
            (adjd_ref, fsd_ref, fdd_ref, whd_ref, bd_ref)):
        e = _leaky(fs_ref[...] + fd_ref[...])
        h3 = _masked_att_agg(a_ref[...], e, wh_ref[...])
        wh = score(h3)
        m = jnp.maximum(wh, wz)
        p1 = jnp.exp(wh - m)
        p2 = jnp.exp(wz - m)
        s = p1 + p2
        b1 = p1 / s
        b2 = p2 / s
        b_ref[...] = jnp.concatenate([b1, b2], axis=1)
        embs.append(b1 * h3 + b2 * zb)

    w1, w2, w3 = score(embs[0]), score(embs[1]), score(embs[2])
    m = jnp.maximum(jnp.maximum(w1, w2), w3)
    p1 = jnp.exp(w1 - m)
    p2 = jnp.exp(w2 - m)
    p3 = jnp.exp(w3 - m)
    s = p1 + p2 + p3
    emb_ref[...] = (p1 / s) * embs[0] + (p2 / s) * embs[1] + (p3 / s) * embs[2]


def _g3k5(adj, adj_diff, f3, Wh3, z, Wp1, bp1_row, wp2_row):
    nz = z.shape[1]
    f_specs = []
    f_args = []
    for fs, fd in f3:
        f_specs += [pl.BlockSpec((BM, 1), _rows), pl.BlockSpec((1, N), _const)]
        f_args += [fs, fd.reshape(1, N)]
    return pl.pallas_call(
        _g3k5_body,
        grid=(N // BM,),
        in_specs=[
            pl.BlockSpec((BM, N), _rows),
            pl.BlockSpec((BM, N), _rows),
        ] + f_specs + [
            pl.BlockSpec((N, nz), _const),
            pl.BlockSpec((N, nz), _const),
            pl.BlockSpec((N, nz), _const),
            pl.BlockSpec((BM, nz), _rows),
            pl.BlockSpec((nz, nz), _const),
            pl.BlockSpec((1, nz), _const),
            pl.BlockSpec((1, nz), _const),
        ],
        out_specs=[
            pl.BlockSpec((BM, nz), _rows),
            pl.BlockSpec((BM, 2), _rows),
            pl.BlockSpec((BM, 2), _rows),
            pl.BlockSpec((BM, 2), _rows),
        ],
        out_shape=[
            jax.ShapeDtypeStruct((N, nz), jnp.float32),
            jax.ShapeDtypeStruct((N, 2), jnp.float32),
            jax.ShapeDtypeStruct((N, 2), jnp.float32),
            jax.ShapeDtypeStruct((N, 2), jnp.float32),
        ],
    )(adj, adj_diff, *f_args, Wh3[0], Wh3[1], Wh3[2], z, Wp1,
      bp1_row, wp2_row)


def kernel(x, adj, adj_knn, adj_diff, We1, be1, We2, be2, Wz, bz, Wd1, bd1,
           Wd2, bd2, Wxb, bxb, Wg1, ag1s, ag1d, Wg2, ag2s, ag2d, Wg3, ag3s,
           ag3d, Wp1, bp1, Wp2):
    enc_h1, Wh1, f1s, f1d = _k1(x, We1, be1.reshape(1, -1), Wg1,
                                ag1s.reshape(1, -1), ag1d.reshape(1, -1))

    enc_h2, z, x_bar = _k2(enc_h1, We2, be2.reshape(1, -1), Wz,
                           bz.reshape(1, -1), Wd1, bd1.reshape(1, -1),
                           Wd2, bd2.reshape(1, -1), Wxb, bxb.reshape(1, -1))

    wh2a, wh2k, wh2d, fsa, fda, fsk, fdk, fsd, fdd = _g1m2(
        adj, adj_knn, adj_diff, f1s, f1d.reshape(1, N), Wh1, enc_h1, Wg2,
        ag2s.reshape(1, -1), ag2d.reshape(1, -1))

    wh3a, wh3k, wh3d, osa, oda, osk, odk, osd, odd = _g2m3(
        adj, adj_knn, adj_diff,
        ((fsa, fda), (fsk, fdk), (fsd, fdd)),
        (wh2a, wh2k, wh2d), enc_h2, Wg3,
        ag3s.reshape(1, -1), ag3d.reshape(1, -1))

    emb_last, b_adj, b_knn, b_diff = _g3k5(
        adj, adj_diff,
        ((osa, oda), (osk, odk), (osd, odd)),
        (wh3a, wh3k, wh3d), z, Wp1, bp1.reshape(1, -1), Wp2.reshape(1, -1))

    return (emb_last,
            b_adj.reshape(N, 2, 1),
            b_knn.reshape(N, 2, 1),
            b_diff.reshape(N, 2, 1),
            x_bar)
